# baseline (speedup 1.0000x reference)
import functools

import jax
import jax.numpy as jnp
from jax import lax
from jax.experimental import pallas as pl
from jax.experimental.pallas import tpu as pltpu
from jax.experimental.pallas import tpu_sc as plsc

VOCAB = 100000
DIM = 64
B, H, W, L = 1024, 4, 4, 20
CELLS = B * H * W
NW = 32
CELLS_PER_W = CELLS // NW
GROUPS = CELLS_PER_W // 16
IDX_PER_W = CELLS_PER_W * L
COLS_BLK = 20480


def _matvec_body(w_ref, t_ref, b_ref, o_ref):
    o_ref[...] = (
        lax.dot_general(
            w_ref[...], t_ref[...],
            dimension_numbers=(((1,), (1,)), ((), ())),
            preferred_element_type=jnp.float32,
            precision=jax.lax.Precision.HIGHEST,
        )
        + b_ref[0, 0]
    ).reshape(COLS_BLK)


def _project_table(table, pred_w, pred_b):
    pred_w = pred_w.astype(jnp.float32)
    b20 = (pred_b.astype(jnp.float32) / jnp.float32(L)).reshape(1, 1)
    grid = (VOCAB + COLS_BLK - 1) // COLS_BLK
    return pl.pallas_call(
        _matvec_body,
        grid=(grid,),
        in_specs=[
            pl.BlockSpec((1, DIM), lambda i: (jnp.int32(0), jnp.int32(0))),
            pl.BlockSpec((COLS_BLK, DIM), lambda i: (i, jnp.int32(0))),
            pl.BlockSpec((1, 1), lambda i: (jnp.int32(0), jnp.int32(0))),
        ],
        out_specs=pl.BlockSpec((COLS_BLK,), lambda i: (i,)),
        out_shape=jax.ShapeDtypeStruct((VOCAB,), jnp.float32),
    )(pred_w, table, b20)


@functools.lru_cache(maxsize=1)
def _make_sc_gather_sum():
    mesh = plsc.VectorSubcoreMesh(core_axis_name="c", subcore_axis_name="s")

    @functools.partial(
        pl.kernel,
        mesh=mesh,
        out_type=jax.ShapeDtypeStruct((CELLS,), jnp.float32),
        scratch_types=[
            pltpu.VMEM((VOCAB,), jnp.float32),
            pltpu.VMEM((IDX_PER_W,), jnp.int32),
            pltpu.VMEM((CELLS_PER_W,), jnp.float32),
            pltpu.SemaphoreType.DMA,
            pltpu.SemaphoreType.DMA,
        ],
        compiler_params=pltpu.CompilerParams(needs_layout_passes=False),
    )
    def _sc_gather_sum(p_hbm, idx_hbm, out_hbm, p_v, idx_v, acc_v, sem_p, sem_i):
        wid = lax.axis_index("s") * 2 + lax.axis_index("c")
        cp_p = pltpu.async_copy(p_hbm, p_v, sem_p)
        cp_i = pltpu.async_copy(
            idx_hbm.at[pl.ds(wid * IDX_PER_W, IDX_PER_W)], idx_v, sem_i
        )
        cp_i.wait()
        cp_p.wait()
        iota20 = lax.iota(jnp.int32, 16) * jnp.int32(L)

        @plsc.parallel_loop(
            jnp.int32(0), jnp.int32(GROUPS), step=jnp.int32(1), unroll=4
        )
        def body(c):
            base = c * jnp.int32(16 * L)
            vals = []
            for l in range(L):
                pos = iota20 + (base + jnp.int32(l))
                iv = plsc.load_gather(idx_v, [pos])
                vals.append(plsc.load_gather(p_v, [iv]))
            while len(vals) > 1:
                vals = [a + b for a, b in zip(vals[::2], vals[1::2])] + (
                    [vals[-1]] if len(vals) % 2 else []
                )
            acc_v[pl.ds(c * jnp.int32(16), 16)] = vals[0]

        pltpu.sync_copy(acc_v, out_hbm.at[pl.ds(wid * CELLS_PER_W, CELLS_PER_W)])

    return _sc_gather_sum


def kernel(x, table, pred_w, pred_b):
    p = _project_table(table, pred_w, pred_b)
    xi = x.reshape(CELLS * L).astype(jnp.int32)
    out_flat = _make_sc_gather_sum()(p, xi)
    return out_flat.reshape(B, H, W).astype(jnp.float64)


# speedup vs baseline: 1.8926x; 1.8926x over previous
import functools

import jax
import jax.numpy as jnp
from jax import lax
from jax.experimental import pallas as pl
from jax.experimental.pallas import tpu as pltpu
from jax.experimental.pallas import tpu_sc as plsc

VOCAB = 100000
DIM = 64
B, H, W, L = 1024, 4, 4, 20
CELLS = B * H * W
NW = 32
CELLS_PER_W = CELLS // NW
GROUPS = CELLS_PER_W // 16
IDX_PER_W = CELLS_PER_W * L
COLS_BLK = 25600


def _matvec_body(w_ref, t_ref, b_ref, o_ref):
    o_ref[...] = (
        jnp.dot(w_ref[...], t_ref[...], preferred_element_type=jnp.float32,
                precision=jax.lax.Precision.HIGHEST)
        + b_ref[0, 0]
    ).reshape(COLS_BLK)


def _project_table(table, pred_w, pred_b):
    tt = table.T
    pred_w = pred_w.astype(jnp.float32)
    b20 = (pred_b.astype(jnp.float32) / jnp.float32(L)).reshape(1, 1)
    grid = (VOCAB + COLS_BLK - 1) // COLS_BLK
    return pl.pallas_call(
        _matvec_body,
        grid=(grid,),
        in_specs=[
            pl.BlockSpec((1, DIM), lambda i: (jnp.int32(0), jnp.int32(0))),
            pl.BlockSpec((DIM, COLS_BLK), lambda i: (jnp.int32(0), i)),
            pl.BlockSpec((1, 1), lambda i: (jnp.int32(0), jnp.int32(0))),
        ],
        out_specs=pl.BlockSpec((COLS_BLK,), lambda i: (i,)),
        out_shape=jax.ShapeDtypeStruct((VOCAB,), jnp.float32),
    )(pred_w, tt, b20)


@functools.lru_cache(maxsize=1)
def _make_sc_gather_sum():
    mesh = plsc.VectorSubcoreMesh(core_axis_name="c", subcore_axis_name="s")

    @functools.partial(
        pl.kernel,
        mesh=mesh,
        out_type=jax.ShapeDtypeStruct((CELLS,), jnp.float32),
        scratch_types=[
            pltpu.VMEM((VOCAB,), jnp.float32),
            pltpu.VMEM((IDX_PER_W,), jnp.int32),
            pltpu.VMEM((CELLS_PER_W,), jnp.float32),
            pltpu.SemaphoreType.DMA,
            pltpu.SemaphoreType.DMA,
        ],
        compiler_params=pltpu.CompilerParams(needs_layout_passes=False),
    )
    def _sc_gather_sum(p_hbm, idx_hbm, out_hbm, p_v, idx_v, acc_v, sem_p, sem_i):
        wid = lax.axis_index("s") * 2 + lax.axis_index("c")
        cp_p = pltpu.async_copy(p_hbm, p_v, sem_p)
        cp_i = pltpu.async_copy(
            idx_hbm.at[pl.ds(wid * IDX_PER_W, IDX_PER_W)], idx_v, sem_i
        )
        cp_i.wait()
        cp_p.wait()
        iota20 = lax.iota(jnp.int32, 16) * jnp.int32(L)

        @plsc.parallel_loop(
            jnp.int32(0), jnp.int32(GROUPS), step=jnp.int32(1), unroll=4
        )
        def body(c):
            base = c * jnp.int32(16 * L)
            vals = []
            for l in range(L):
                pos = iota20 + (base + jnp.int32(l))
                iv = plsc.load_gather(idx_v, [pos])
                vals.append(plsc.load_gather(p_v, [iv]))
            while len(vals) > 1:
                vals = [a + b for a, b in zip(vals[::2], vals[1::2])] + (
                    [vals[-1]] if len(vals) % 2 else []
                )
            acc_v[pl.ds(c * jnp.int32(16), 16)] = vals[0]

        pltpu.sync_copy(acc_v, out_hbm.at[pl.ds(wid * CELLS_PER_W, CELLS_PER_W)])

    return _sc_gather_sum


def kernel(x, table, pred_w, pred_b):
    p = _project_table(table, pred_w, pred_b)
    xi = x.reshape(CELLS * L).astype(jnp.int32)
    out_flat = _make_sc_gather_sum()(p, xi)
    return out_flat.reshape(B, H, W).astype(jnp.float64)
